# baseline (speedup 1.0000x reference)
import jax
import jax.numpy as jnp
from jax import lax
from jax.experimental import pallas as pl
from jax.experimental.pallas import tpu as pltpu

SRC = 10000
DST = 10000
E = 160000
B = 2
D = 256
NP = 10240
SUB = 512
NSB = NP // SUB
EC = 320
NCH = E // EC
EPS = 1e-8
NH = 128


def _tc_body(srcs, dsts, ws, x, out, nacc):
    j = pl.program_id(0)

    @pl.when(j == 0)
    def _():
        out[...] = jnp.zeros_like(out)
        nacc[...] = jnp.zeros_like(nacc)

    src = srcs[0, 0, :]
    dst = dsts[0, 0, :]
    w = ws[0, 0, :]
    wb = jnp.broadcast_to(
        w.astype(jnp.bfloat16)[:, None], (EC, NH))

    for b in range(B):
        y = jnp.zeros((EC, D), jnp.float32)
        for sb in range(NSB):
            ids = lax.broadcasted_iota(jnp.int32, (EC, SUB), 1) + sb * SUB
            soh = (src[:, None] == ids).astype(jnp.bfloat16)
            y = y + jnp.dot(soh, x[b, pl.ds(sb * SUB, SUB), :],
                            preferred_element_type=jnp.float32)
        yb = (y * w[:, None]).astype(jnp.bfloat16)
        for db in range(NSB):
            rid = lax.broadcasted_iota(jnp.int32, (SUB, EC), 0) + db * SUB
            doh = (dst[None, :] == rid).astype(jnp.bfloat16)
            out[b, pl.ds(db * SUB, SUB), :] += jnp.dot(
                doh, yb, preferred_element_type=jnp.float32)
            if b == 0:
                nacc[pl.ds(db * SUB, SUB), :] += jnp.dot(
                    doh, wb, preferred_element_type=jnp.float32)

    @pl.when(j == NCH - 1)
    def _():
        for nb in range(NSB):
            inv = 1.0 / (nacc[pl.ds(nb * SUB, SUB), :] + EPS)
            for b in range(B):
                out[b, pl.ds(nb * SUB, SUB), pl.ds(0, NH)] *= inv
                out[b, pl.ds(nb * SUB, SUB), pl.ds(NH, NH)] *= inv


@jax.jit
def kernel(x, edge_index, weights):
    xp = jnp.zeros((B, NP, D), jnp.bfloat16)
    xp = xp.at[:, :SRC, :].set(x.astype(jnp.bfloat16))
    srcs = edge_index[0].reshape(NCH, 1, EC)
    dsts = edge_index[1].reshape(NCH, 1, EC)
    ws = weights.reshape(NCH, 1, EC)

    out = pl.pallas_call(
        _tc_body,
        grid=(NCH,),
        in_specs=[
            pl.BlockSpec((1, 1, EC), lambda j: (j, 0, 0)),
            pl.BlockSpec((1, 1, EC), lambda j: (j, 0, 0)),
            pl.BlockSpec((1, 1, EC), lambda j: (j, 0, 0)),
            pl.BlockSpec((B, NP, D), lambda j: (0, 0, 0)),
        ],
        out_specs=pl.BlockSpec((B, NP, D), lambda j: (0, 0, 0)),
        out_shape=jax.ShapeDtypeStruct((B, NP, D), jnp.float32),
        scratch_shapes=[pltpu.VMEM((NP, NH), jnp.float32)],
        compiler_params=pltpu.CompilerParams(
            dimension_semantics=("arbitrary",),
        ),
    )(srcs, dsts, ws, xp)
    return out[:, :DST, :]


# speedup vs baseline: 1.2804x; 1.2804x over previous
import jax
import jax.numpy as jnp
from jax import lax
from jax.experimental import pallas as pl
from jax.experimental.pallas import tpu as pltpu

SRC = 10000
DST = 10000
E = 160000
B = 2
D = 256
NP = 10240
SUB = 512
NSB = NP // SUB
EC = 320
NCH = E // EC
EPS = 1e-8
NH = 128
BD = B * D


def _tc_body(srcs, dsts, ws, x, out):
    j = pl.program_id(0)

    @pl.when(j == 0)
    def _():
        out[...] = jnp.zeros_like(out)

    src = srcs[0, 0, :]
    dst = dsts[0, 0, :]
    w = ws[0, 0, :]
    wb = jnp.broadcast_to(w.astype(jnp.bfloat16)[:, None], (EC, NH))

    y = jnp.zeros((EC, BD), jnp.float32)
    for sb in range(NSB):
        ids = lax.broadcasted_iota(jnp.int32, (EC, SUB), 1) + sb * SUB
        soh = (src[:, None] == ids).astype(jnp.bfloat16)
        y = y + jnp.dot(soh, x[pl.ds(sb * SUB, SUB), :],
                        preferred_element_type=jnp.float32)
    yb = (y * w[:, None]).astype(jnp.bfloat16)

    for db in range(NSB):
        rid = lax.broadcasted_iota(jnp.int32, (SUB, EC), 0) + db * SUB
        doh = (dst[None, :] == rid).astype(jnp.bfloat16)
        out[pl.ds(db * SUB, SUB), pl.ds(0, BD)] += jnp.dot(
            doh, yb, preferred_element_type=jnp.float32)
        out[pl.ds(db * SUB, SUB), pl.ds(BD, NH)] += jnp.dot(
            doh, wb, preferred_element_type=jnp.float32)

    @pl.when(j == NCH - 1)
    def _():
        for nb in range(NSB):
            inv = 1.0 / (out[pl.ds(nb * SUB, SUB), pl.ds(BD, NH)] + EPS)
            for q in range(BD // NH):
                out[pl.ds(nb * SUB, SUB), pl.ds(q * NH, NH)] *= inv


@jax.jit
def kernel(x, edge_index, weights):
    xp = jnp.zeros((NP, BD), jnp.bfloat16)
    xp = xp.at[:SRC, :].set(
        x.transpose(1, 0, 2).reshape(SRC, BD).astype(jnp.bfloat16))
    srcs = edge_index[0].reshape(NCH, 1, EC)
    dsts = edge_index[1].reshape(NCH, 1, EC)
    ws = weights.reshape(NCH, 1, EC)

    out = pl.pallas_call(
        _tc_body,
        grid=(NCH,),
        in_specs=[
            pl.BlockSpec((1, 1, EC), lambda j: (j, 0, 0)),
            pl.BlockSpec((1, 1, EC), lambda j: (j, 0, 0)),
            pl.BlockSpec((1, 1, EC), lambda j: (j, 0, 0)),
            pl.BlockSpec((NP, BD), lambda j: (0, 0)),
        ],
        out_specs=pl.BlockSpec((NP, BD + NH), lambda j: (0, 0)),
        out_shape=jax.ShapeDtypeStruct((NP, BD + NH), jnp.float32),
        compiler_params=pltpu.CompilerParams(
            dimension_semantics=("arbitrary",),
        ),
    )(srcs, dsts, ws, xp)
    return out[:DST, :BD].reshape(DST, B, D).transpose(1, 0, 2)


# speedup vs baseline: 1.5539x; 1.2136x over previous
import jax
import jax.numpy as jnp
from jax import lax
from jax.experimental import pallas as pl
from jax.experimental.pallas import tpu as pltpu

SRC = 10000
DST = 10000
E = 160000
B = 2
D = 256
NP = 10240
SUB = 512
NSB = NP // SUB
EC = 640
NCH = E // EC
EPS = 1e-8
NH = 128
BD = B * D


def _tc_body(srcs, dsts, ws, x, out):
    j = pl.program_id(0)

    @pl.when(j == 0)
    def _():
        out[...] = jnp.zeros_like(out)

    src = srcs[0, 0, :]
    dst = dsts[0, 0, :]
    w = ws[0, 0, :]
    wb = jnp.broadcast_to(w.astype(jnp.bfloat16)[:, None], (EC, NH))

    y = jnp.zeros((EC, BD), jnp.float32)
    for sb in range(NSB):
        ids = lax.broadcasted_iota(jnp.int32, (EC, SUB), 1) + sb * SUB
        soh = (src[:, None] == ids).astype(jnp.bfloat16)
        y = y + jnp.dot(soh, x[pl.ds(sb * SUB, SUB), :],
                        preferred_element_type=jnp.float32)
    yb = (y * w[:, None]).astype(jnp.bfloat16)

    for db in range(NSB):
        rid = lax.broadcasted_iota(jnp.int32, (SUB, EC), 0) + db * SUB
        doh = (dst[None, :] == rid).astype(jnp.bfloat16)
        out[pl.ds(db * SUB, SUB), pl.ds(0, BD)] += jnp.dot(
            doh, yb, preferred_element_type=jnp.float32)
        out[pl.ds(db * SUB, SUB), pl.ds(BD, NH)] += jnp.dot(
            doh, wb, preferred_element_type=jnp.float32)

    @pl.when(j == NCH - 1)
    def _():
        for nb in range(NSB):
            inv = 1.0 / (out[pl.ds(nb * SUB, SUB), pl.ds(BD, NH)] + EPS)
            for q in range(BD // NH):
                out[pl.ds(nb * SUB, SUB), pl.ds(q * NH, NH)] *= inv


@jax.jit
def kernel(x, edge_index, weights):
    xp = jnp.zeros((NP, BD), jnp.bfloat16)
    xp = xp.at[:SRC, :].set(
        x.transpose(1, 0, 2).reshape(SRC, BD).astype(jnp.bfloat16))
    srcs = edge_index[0].reshape(NCH, 1, EC)
    dsts = edge_index[1].reshape(NCH, 1, EC)
    ws = weights.reshape(NCH, 1, EC)

    out = pl.pallas_call(
        _tc_body,
        grid=(NCH,),
        in_specs=[
            pl.BlockSpec((1, 1, EC), lambda j: (j, 0, 0)),
            pl.BlockSpec((1, 1, EC), lambda j: (j, 0, 0)),
            pl.BlockSpec((1, 1, EC), lambda j: (j, 0, 0)),
            pl.BlockSpec((NP, BD), lambda j: (0, 0)),
        ],
        out_specs=pl.BlockSpec((NP, BD + NH), lambda j: (0, 0)),
        out_shape=jax.ShapeDtypeStruct((NP, BD + NH), jnp.float32),
        compiler_params=pltpu.CompilerParams(
            dimension_semantics=("arbitrary",),
        ),
    )(srcs, dsts, ws, xp)
    return out[:DST, :BD].reshape(DST, B, D).transpose(1, 0, 2)
